# baseline (speedup 1.0000x reference)
import functools

import jax
import jax.numpy as jnp
from jax import lax
from jax.experimental import pallas as pl
from jax.experimental.pallas import tpu as pltpu
from jax.experimental.pallas import tpu_sc as plsc

DIM = 64
SCALE = 8.0

_NC = 2
_NS = 16
_NW = _NC * _NS

_C = 128
_NBUF = 4


@jax.jit
def _emb_call(xt3, lut):
    nj, ncb, _ = xt3.shape
    nunit = nj * ncb
    upw = nunit // _NW
    mesh = plsc.VectorSubcoreMesh(core_axis_name="c", subcore_axis_name="s")

    @functools.partial(
        pl.kernel,
        out_type=jax.ShapeDtypeStruct((nj, DIM // 8, ncb, 8, _C), jnp.float32),
        mesh=mesh,
        scratch_types=[
            pltpu.VMEM((_NBUF, _C), jnp.int32),
            pltpu.VMEM((_NBUF, _C, DIM), jnp.float32),
            pltpu.VMEM((_NBUF, DIM, _C), jnp.float32),
            pltpu.SemaphoreType.DMA((_NBUF,)),
            pltpu.SemaphoreType.DMA((_NBUF,)),
            pltpu.SemaphoreType.DMA((_NBUF,)),
        ],
        compiler_params=pltpu.CompilerParams(
            use_tc_tiling_on_sc=False, needs_layout_passes=False),
    )
    def emb(x_hbm, lut_hbm, out_hbm, idx_v, rows_v, tr_v, isem, gsem, osem):
        wid = lax.axis_index("s") * _NC + lax.axis_index("c")
        ubase = wid * upw
        iota = lax.iota(jnp.int32, 16)

        def unit_jc(u):
            ug = ubase + u
            return ug // ncb, ug % ncb

        def idx_copy(u, b):
            j, c = unit_jc(u)
            return pltpu.make_async_copy(x_hbm.at[j, c], idx_v.at[b], isem.at[b])

        def gather(b):
            return pltpu.make_async_copy(
                lut_hbm.at[idx_v.at[b]], rows_v.at[b], gsem.at[b])

        def out_copy(u, b, r):
            j, c = unit_jc(u)
            return pltpu.make_async_copy(
                tr_v.at[b, pl.ds(r * 8, 8)], out_hbm.at[j, r, c], osem.at[b])

        for k in range(_NBUF - 1):
            idx_copy(k, k).start()
        idx_copy(0, 0).wait()
        gather(0).start()

        @pl.loop(0, upw, step=_NBUF)
        def outer(u0):
            for b in range(_NBUF):
                u = u0 + b

                @pl.when(u + _NBUF - 1 < upw)
                def _():
                    idx_copy(u + _NBUF - 1, (b + _NBUF - 1) % _NBUF).start()

                @pl.when(u + 1 < upw)
                def _():
                    nb = (b + 1) % _NBUF
                    idx_copy(u + 1, nb).wait()
                    gather(nb).start()

                gather(b).wait()

                @pl.when(u >= _NBUF)
                def _():
                    for r in range(DIM // 8):
                        out_copy(u - _NBUF, b, r).wait()

                @plsc.parallel_loop(0, _C, unroll=8)
                def rloop(r):
                    for k in range(DIM // 16):
                        v = rows_v[b, r, pl.ds(k * 16, 16)]
                        plsc.store_scatter(
                            tr_v.at[b], [iota + (k * 16), jnp.full((16,), r, jnp.int32)],
                            v * SCALE)

                for r in range(DIM // 8):
                    out_copy(u, b, r).start()

        for b in range(_NBUF):
            for r in range(DIM // 8):
                out_copy(upw - _NBUF + b, b, r).wait()

    return emb(xt3, lut)


def kernel(X, lut):
    s0, s1 = X.shape
    xt3 = jnp.transpose(X).astype(jnp.int32).reshape(s1, s0 // _C, _C)
    out5 = _emb_call(xt3, lut)
    out = out5.transpose(2, 4, 0, 1, 3).reshape(s0, s1, DIM)
    return out


# speedup vs baseline: 1.6626x; 1.6626x over previous
import functools

import jax
import jax.numpy as jnp
from jax import lax
from jax.experimental import pallas as pl
from jax.experimental.pallas import tpu as pltpu
from jax.experimental.pallas import tpu_sc as plsc

DIM = 64
SCALE = 8.0

_NC = 2
_NS = 16
_NW = _NC * _NS

_C = 128
_NBUF = 4


@jax.jit
def _emb_call(xt3, lut):
    nj, ncb, _ = xt3.shape
    nunit = nj * ncb
    upw = nunit // _NW
    mesh = plsc.VectorSubcoreMesh(core_axis_name="c", subcore_axis_name="s")

    @functools.partial(
        pl.kernel,
        out_type=jax.ShapeDtypeStruct((nj, DIM // 8, ncb, 8, _C), jnp.float32),
        mesh=mesh,
        scratch_types=[
            pltpu.VMEM((_NBUF, _C), jnp.int32),
            pltpu.VMEM((_NBUF, _C, DIM), jnp.float32),
            pltpu.VMEM((_NBUF, DIM, _C + 1), jnp.float32),
            pltpu.SemaphoreType.DMA((_NBUF,)),
            pltpu.SemaphoreType.DMA((_NBUF,)),
            pltpu.SemaphoreType.DMA((_NBUF,)),
        ],
        compiler_params=pltpu.CompilerParams(
            use_tc_tiling_on_sc=False, needs_layout_passes=False),
    )
    def emb(x_hbm, lut_hbm, out_hbm, idx_v, rows_v, tr_v, isem, gsem, osem):
        wid = lax.axis_index("s") * _NC + lax.axis_index("c")
        ubase = wid * upw
        iota = lax.iota(jnp.int32, 16)

        def unit_jc(u):
            ug = ubase + u
            return ug // ncb, ug % ncb

        def idx_copy(u, b):
            j, c = unit_jc(u)
            return pltpu.make_async_copy(x_hbm.at[j, c], idx_v.at[b], isem.at[b])

        def gather(b):
            return pltpu.make_async_copy(
                lut_hbm.at[idx_v.at[b]], rows_v.at[b], gsem.at[b])

        def out_copy(u, b, r):
            j, c = unit_jc(u)
            return pltpu.make_async_copy(
                tr_v.at[b, pl.ds(r * 8, 8), pl.ds(0, _C)],
                out_hbm.at[j, r, c], osem.at[b])

        for k in range(_NBUF - 1):
            idx_copy(k, k).start()
        idx_copy(0, 0).wait()
        gather(0).start()

        @pl.loop(0, upw, step=_NBUF)
        def outer(u0):
            for b in range(_NBUF):
                u = u0 + b

                @pl.when(u + _NBUF - 1 < upw)
                def _():
                    idx_copy(u + _NBUF - 1, (b + _NBUF - 1) % _NBUF).start()

                @pl.when(u + 1 < upw)
                def _():
                    nb = (b + 1) % _NBUF
                    idx_copy(u + 1, nb).wait()
                    gather(nb).start()

                gather(b).wait()

                @pl.when(u >= _NBUF)
                def _():
                    for r in range(DIM // 8):
                        out_copy(u - _NBUF, b, r).wait()

                @plsc.parallel_loop(0, _C, unroll=8)
                def rloop(r):
                    col = jnp.full((16,), r, jnp.int32)
                    for k in range(DIM // 16):
                        v = rows_v[b, r, pl.ds(k * 16, 16)]
                        plsc.store_scatter(
                            tr_v.at[b], [iota + (k * 16), col], v * SCALE)

                for r in range(DIM // 8):
                    out_copy(u, b, r).start()

        for b in range(_NBUF):
            for r in range(DIM // 8):
                out_copy(upw - _NBUF + b, b, r).wait()

    return emb(xt3, lut)


def kernel(X, lut):
    s0, s1 = X.shape
    xt3 = jnp.transpose(X).astype(jnp.int32).reshape(s1, s0 // _C, _C)
    out5 = _emb_call(xt3, lut)
    out = out5.transpose(2, 4, 0, 1, 3).reshape(s0, s1, DIM)
    return out


# speedup vs baseline: 1.8810x; 1.1313x over previous
import functools

import jax
import jax.numpy as jnp
from jax import lax
from jax.experimental import pallas as pl
from jax.experimental.pallas import tpu as pltpu
from jax.experimental.pallas import tpu_sc as plsc

DIM = 64
SCALE = 8.0

_NC = 2
_NS = 16
_NW = _NC * _NS

_C = 128
_NBUF = 4


@jax.jit
def _tilegrid_call(lut_t):
    ndim, vocab = lut_t.shape
    vp = (vocab + _C - 1) // _C
    mesh = plsc.VectorSubcoreMesh(core_axis_name="c", subcore_axis_name="s")
    tpw = (vp + _NW - 1) // _NW
    nb = 8

    @functools.partial(
        pl.kernel,
        out_type=jax.ShapeDtypeStruct((vp, ndim, _C), jnp.float32),
        mesh=mesh,
        scratch_types=[
            pltpu.VMEM((nb, ndim, _C), jnp.float32),
            pltpu.SemaphoreType.DMA((nb,)),
            pltpu.SemaphoreType.DMA((nb,)),
        ],
        compiler_params=pltpu.CompilerParams(
            use_tc_tiling_on_sc=True, needs_layout_passes=False),
    )
    def grid(lut_hbm, t5_hbm, st_v, isem, osem):
        wid = lax.axis_index("s") * _NC + lax.axis_index("c")

        def cid(t):
            return wid + t * _NW

        def stripe_in(t, b):
            return pltpu.make_async_copy(
                lut_hbm.at[:, pl.ds(cid(t) * _C, _C)], st_v.at[b], isem.at[b])

        def stripe_out(t, b):
            return pltpu.make_async_copy(
                st_v.at[b], t5_hbm.at[cid(t)], osem.at[b])

        def live(t):
            return cid(t) < vp

        for k in range(nb - 1):
            @pl.when(live(k))
            def _():
                stripe_in(k, k).start()

        @pl.loop(0, tpw, step=nb)
        def outer(t0):
            for b in range(nb):
                t = t0 + b

                @pl.when(live(t + nb - 1))
                def _():
                    @pl.when(t + nb - 1 >= nb)
                    def _():
                        stripe_out(t - 1, (b + nb - 1) % nb).wait()
                    stripe_in(t + nb - 1, (b + nb - 1) % nb).start()

                @pl.when(live(t))
                def _():
                    stripe_in(t, b).wait()
                    stripe_out(t, b).start()

        for s in range(tpw - nb - 1, tpw):
            @pl.when(live(s) & jnp.logical_not(live(s + nb)))
            def _():
                stripe_out(s, s % nb).wait()

    return grid(lut_t)


@jax.jit
def _rowmajor_call(t5):
    vp, ndim, _ = t5.shape
    mesh = plsc.VectorSubcoreMesh(core_axis_name="c", subcore_axis_name="s")
    tpw = (vp + _NW - 1) // _NW

    @functools.partial(
        pl.kernel,
        out_type=jax.ShapeDtypeStruct((vp * _C, ndim), jnp.float32),
        mesh=mesh,
        scratch_types=[
            pltpu.VMEM((_NBUF, ndim, _C), jnp.float32),
            pltpu.VMEM((_NBUF, _C, ndim + 1), jnp.float32),
            pltpu.SemaphoreType.DMA((_NBUF,)),
            pltpu.SemaphoreType.DMA((_NBUF,)),
        ],
        compiler_params=pltpu.CompilerParams(
            use_tc_tiling_on_sc=False, needs_layout_passes=False),
    )
    def rowm(t5_hbm, rm_hbm, in_v, tr_v, isem, osem):
        wid = lax.axis_index("s") * _NC + lax.axis_index("c")
        iota = lax.iota(jnp.int32, 16)

        def cid(t):
            return wid + t * _NW

        def stripe_in(t, b):
            return pltpu.make_async_copy(
                t5_hbm.at[cid(t)], in_v.at[b], isem.at[b])

        def stripe_out(t, b):
            return pltpu.make_async_copy(
                tr_v.at[b, :, pl.ds(0, ndim)],
                rm_hbm.at[pl.ds(cid(t) * _C, _C)], osem.at[b])

        def live(t):
            return cid(t) < vp

        for k in range(_NBUF - 1):
            @pl.when(live(k))
            def _():
                stripe_in(k, k).start()

        @pl.loop(0, tpw, step=_NBUF)
        def outer(t0):
            for b in range(_NBUF):
                t = t0 + b

                @pl.when(live(t + _NBUF - 1))
                def _():
                    stripe_in(t + _NBUF - 1, (b + _NBUF - 1) % _NBUF).start()

                @pl.when(live(t))
                def _():
                    stripe_in(t, b).wait()

                    @pl.when(t >= _NBUF)
                    def _():
                        stripe_out(t - _NBUF, b).wait()

                    @plsc.parallel_loop(0, ndim, unroll=8)
                    def dloop(d):
                        col = jnp.full((16,), d, jnp.int32)
                        for k in range(_C // 16):
                            v = in_v[b, d, pl.ds(k * 16, 16)]
                            plsc.store_scatter(
                                tr_v.at[b], [iota + (k * 16), col], v * SCALE)

                    stripe_out(t, b).start()

        for s in range(tpw - _NBUF - 1, tpw):
            @pl.when(live(s) & jnp.logical_not(live(s + _NBUF)))
            def _():
                stripe_out(s, s % _NBUF).wait()

    return rowm(t5)


@jax.jit
def _emb_call(xt3, lut):
    nj, ncb, _ = xt3.shape
    nunit = nj * ncb
    upw = nunit // _NW
    mesh = plsc.VectorSubcoreMesh(core_axis_name="c", subcore_axis_name="s")

    @functools.partial(
        pl.kernel,
        out_type=jax.ShapeDtypeStruct((nj, DIM // 8, ncb, 8, _C), jnp.float32),
        mesh=mesh,
        scratch_types=[
            pltpu.VMEM((_NBUF, _C), jnp.int32),
            pltpu.VMEM((_NBUF, _C, DIM), jnp.float32),
            pltpu.VMEM((_NBUF, DIM, _C + 1), jnp.float32),
            pltpu.SemaphoreType.DMA((_NBUF,)),
            pltpu.SemaphoreType.DMA((_NBUF,)),
            pltpu.SemaphoreType.DMA((_NBUF,)),
        ],
        compiler_params=pltpu.CompilerParams(
            use_tc_tiling_on_sc=False, needs_layout_passes=False),
    )
    def emb(x_hbm, lut_hbm, out_hbm, idx_v, rows_v, tr_v, isem, gsem, osem):
        wid = lax.axis_index("s") * _NC + lax.axis_index("c")
        ubase = wid * upw
        iota = lax.iota(jnp.int32, 16)

        def unit_jc(u):
            ug = ubase + u
            return ug // ncb, ug % ncb

        def idx_copy(u, b):
            j, c = unit_jc(u)
            return pltpu.make_async_copy(x_hbm.at[j, c], idx_v.at[b], isem.at[b])

        def gather(b):
            return pltpu.make_async_copy(
                lut_hbm.at[idx_v.at[b]], rows_v.at[b], gsem.at[b])

        def out_copy(u, b, r):
            j, c = unit_jc(u)
            return pltpu.make_async_copy(
                tr_v.at[b, pl.ds(r * 8, 8), pl.ds(0, _C)],
                out_hbm.at[j, r, c], osem.at[b])

        for k in range(_NBUF - 1):
            idx_copy(k, k).start()
        idx_copy(0, 0).wait()
        gather(0).start()

        @pl.loop(0, upw, step=_NBUF)
        def outer(u0):
            for b in range(_NBUF):
                u = u0 + b

                @pl.when(u + _NBUF - 1 < upw)
                def _():
                    idx_copy(u + _NBUF - 1, (b + _NBUF - 1) % _NBUF).start()

                @pl.when(u + 1 < upw)
                def _():
                    nb = (b + 1) % _NBUF
                    idx_copy(u + 1, nb).wait()
                    gather(nb).start()

                gather(b).wait()

                @pl.when(u >= _NBUF)
                def _():
                    for r in range(DIM // 8):
                        out_copy(u - _NBUF, b, r).wait()

                @plsc.parallel_loop(0, _C, unroll=8)
                def rloop(r):
                    col = jnp.full((16,), r, jnp.int32)
                    for k in range(DIM // 16):
                        v = rows_v[b, r, pl.ds(k * 16, 16)]
                        plsc.store_scatter(
                            tr_v.at[b], [iota + (k * 16), col], v)

                for r in range(DIM // 8):
                    out_copy(u, b, r).start()

        for b in range(_NBUF):
            for r in range(DIM // 8):
                out_copy(upw - _NBUF + b, b, r).wait()

    return emb(xt3, lut)


def kernel(X, lut):
    s0, s1 = X.shape
    tbl = _rowmajor_call(_tilegrid_call(jnp.transpose(lut)))
    xt3 = jnp.transpose(X).astype(jnp.int32).reshape(s1, s0 // _C, _C)
    out5 = _emb_call(xt3, tbl)
    out = out5.transpose(2, 4, 0, 1, 3).reshape(s0, s1, DIM)
    return out
